# baseline (speedup 1.0000x reference)
import jax
import jax.numpy as jnp
from jax import lax
from jax.experimental import pallas as pl
from jax.experimental.pallas import tpu as pltpu
from jax.experimental.pallas import tpu_sc as plsc

WORD_DIM = 128
TAG_NUM = 64
TAG_DIM = 32
TAG_PITCH = TAG_DIM + 1
OUT_DIM = WORD_DIM + TAG_DIM
DBLK = OUT_DIM // 8
TAG_SHIFT = 17
WORD_MASK = (1 << TAG_SHIFT) - 1
T_PITCH = 129

NC = 2
NS = 16
NW = NC * NS

CHUNK = 128
KB = 4


def _emb_kernel(comb_hbm, word_table_hbm, tag_pad_hbm,
                out_hbm, comb_v, tag_v, tmp_v, rows_w_v, trans_v, idxl_v,
                sem_g, sem_o):
    n_s = comb_hbm.shape[0]
    wid = lax.axis_index("s") * NC + lax.axis_index("c")
    b0 = wid * CHUNK

    pltpu.sync_copy(comb_hbm.at[:, pl.ds(b0, CHUNK)], comb_v)
    pltpu.sync_copy(tag_pad_hbm, tag_v)

    iota = lax.iota(jnp.int32, 16)
    rvecs = [iota + g * 16 for g in range(8)]
    pvecs = [rvecs[g] * T_PITCH for g in range(8)]

    def extract_idx(s, p):
        for g in range(8):
            cvec = comb_v[s, pl.ds(g * 16, 16)]
            idxl_v[p][pl.ds(g * 16, 16)] = cvec & WORD_MASK

    def build_tag(s, p):
        for g in range(8):
            cvec = comb_v[s, pl.ds(g * 16, 16)]
            off = lax.shift_right_logical(cvec, TAG_SHIFT) * TAG_PITCH
            g16 = g * 16
            for d0 in range(0, TAG_DIM, 4):
                gs = [plsc.load_gather(tag_v, [off + (d0 + i)])
                      for i in range(4)]
                for i in range(4):
                    row = WORD_DIM + d0 + i
                    trans_v[p][row // 8, 0, row % 8, pl.ds(g16, 16)] = gs[i]

    def scatter_word(p):
        def kloop(k0, _):
            k = k0 * KB
            vs = [rows_w_v[p][k + dk, pl.ds(j * 16, 16)]
                  for dk in range(KB) for j in range(8)]
            for dk in range(KB):
                for j in range(8):
                    plsc.store_scatter(
                        tmp_v, [pvecs[j] + (k + dk)], vs[dk * 8 + j])
            return ()
        lax.fori_loop(0, CHUNK // KB, kloop, (), unroll=2)

    def copy_word(p):
        def rloop(r0, _):
            r = r0 * KB
            vs = [tmp_v[pl.ds((r + dr) * T_PITCH + j * 16, 16)]
                  for dr in range(KB) for j in range(8)]
            for dr in range(KB):
                rr = r + dr
                for j in range(8):
                    trans_v[p][rr // 8, 0, rr % 8, pl.ds(j * 16, 16)] = (
                        vs[dr * 8 + j])
            return ()
        lax.fori_loop(0, WORD_DIM // KB, rloop, (), unroll=2)

    def gather_desc(p):
        return pltpu.make_async_copy(
            word_table_hbm.at[idxl_v[p]], rows_w_v[p], sem_g[p])

    def out_descs(s, p):
        return [
            pltpu.make_async_copy(
                trans_v[p],
                out_hbm.at[s, :, pl.ds(wid, 1), :, :],
                sem_o[p])
        ]

    extract_idx(0, 0)
    gather_desc(0).start()

    def body(so, _):
        for b in range(2):
            s = so * 2 + b
            q = 1 - b

            @pl.when(so > 0)
            def _wait_prev_out():
                for c in out_descs(s, b):
                    c.wait()

            @pl.when(s + 1 < n_s)
            def _prefetch_next():
                extract_idx(s + 1, q)
                gather_desc(q).start()

            build_tag(s, b)
            gather_desc(b).wait()
            scatter_word(b)
            copy_word(b)
            for c in out_descs(s, b):
                c.start()
        return ()

    lax.fori_loop(0, n_s // 2, body, (), unroll=False)

    for b in range(2):
        for c in out_descs(n_s - 2 + b, b):
            c.wait()


def kernel(word_ids, tag_ids, word_table, tag_table):
    b, s = word_ids.shape
    comb = (word_ids.astype(jnp.int32)
            | (tag_ids.astype(jnp.int32) << TAG_SHIFT)).T
    tag_pad = jnp.pad(tag_table, ((0, 0), (0, TAG_PITCH - TAG_DIM))).reshape(-1)

    run = pl.kernel(
        _emb_kernel,
        out_type=jax.ShapeDtypeStruct((s, DBLK, NW, 8, CHUNK), jnp.float32),
        mesh=plsc.VectorSubcoreMesh(core_axis_name="c", subcore_axis_name="s"),
        compiler_params=pltpu.CompilerParams(needs_layout_passes=False),
        scratch_types=[
            pltpu.VMEM((s, CHUNK), jnp.int32),
            pltpu.VMEM((TAG_NUM * TAG_PITCH,), jnp.float32),
            pltpu.VMEM((WORD_DIM * T_PITCH,), jnp.float32),
            [pltpu.VMEM((CHUNK, WORD_DIM), jnp.float32) for _ in range(2)],
            [pltpu.VMEM((DBLK, 1, 8, CHUNK), jnp.float32) for _ in range(2)],
            [pltpu.VMEM((CHUNK,), jnp.int32) for _ in range(2)],
            [pltpu.SemaphoreType.DMA for _ in range(2)],
            [pltpu.SemaphoreType.DMA for _ in range(2)],
        ],
    )
    out = run(comb, word_table, tag_pad)
    out = out.transpose(2, 4, 0, 1, 3).reshape(b, s, OUT_DIM)
    return out


# speedup vs baseline: 1.0258x; 1.0258x over previous
import jax
import jax.numpy as jnp
from jax import lax
from jax.experimental import pallas as pl
from jax.experimental.pallas import tpu as pltpu
from jax.experimental.pallas import tpu_sc as plsc

WORD_DIM = 128
TAG_NUM = 64
TAG_DIM = 32
TAG_PITCH = TAG_DIM + 1
OUT_DIM = WORD_DIM + TAG_DIM
DBLK = OUT_DIM // 8
TAG_SHIFT = 17
WORD_MASK = (1 << TAG_SHIFT) - 1

NC = 2
NS = 16
NW = NC * NS

CHUNK = 128


def _emb_kernel(comb_hbm, word_table_hbm, tag_pad_hbm,
                out_hbm, comb_v, tag_v, rows_w_v, trans_v, idxl_v,
                sem_g, sem_o):
    n_s = comb_hbm.shape[0]
    wid = lax.axis_index("s") * NC + lax.axis_index("c")
    b0 = wid * CHUNK

    pltpu.sync_copy(comb_hbm.at[:, pl.ds(b0, CHUNK)], comb_v)
    pltpu.sync_copy(tag_pad_hbm, tag_v)

    iota = lax.iota(jnp.int32, 16)
    cdiags = [lax.rem(iota + t, jnp.full((16,), 16, jnp.int32))
              for t in range(16)]

    def extract_idx(s, p):
        for g in range(8):
            cvec = comb_v[s, pl.ds(g * 16, 16)]
            idxl_v[p][pl.ds(g * 16, 16)] = cvec & WORD_MASK

    def build_tag(s, p):
        for g in range(8):
            cvec = comb_v[s, pl.ds(g * 16, 16)]
            off = lax.shift_right_logical(cvec, TAG_SHIFT) * TAG_PITCH
            g16 = g * 16
            for d0 in range(0, TAG_DIM, 8):
                gs = [plsc.load_gather(tag_v, [off + (d0 + i)])
                      for i in range(8)]
                for i in range(8):
                    trans_v[p][WORD_DIM + d0 + i, pl.ds(g16, 16)] = gs[i]

    def transpose_word(p):
        def kblk(kb, _):
            krow = iota + kb * 16
            for db in range(8):
                d0 = db * 16
                vs = [plsc.load_gather(rows_w_v[p], [krow, cdiags[t] + d0])
                      for t in range(16)]
                for t in range(16):
                    plsc.store_scatter(
                        trans_v[p], [cdiags[t] + d0, krow], vs[t])
            return ()
        lax.fori_loop(0, CHUNK // 16, kblk, (), unroll=1)

    def gather_desc(p):
        return pltpu.make_async_copy(
            word_table_hbm.at[idxl_v[p]], rows_w_v[p], sem_g[p])

    def out_descs(s, p):
        return [
            pltpu.make_async_copy(
                trans_v[p].at[pl.ds(d * 8, 8), :],
                out_hbm.at[s, d, wid, :, :],
                sem_o[p])
            for d in range(DBLK)
        ]

    extract_idx(0, 0)
    gather_desc(0).start()

    def body(so, _):
        for b in range(2):
            s = so * 2 + b
            q = 1 - b

            @pl.when(so > 0)
            def _wait_prev_out():
                for c in out_descs(s, b):
                    c.wait()

            @pl.when(s + 1 < n_s)
            def _prefetch_next():
                extract_idx(s + 1, q)
                gather_desc(q).start()

            build_tag(s, b)
            gather_desc(b).wait()
            transpose_word(b)
            for c in out_descs(s, b):
                c.start()
        return ()

    lax.fori_loop(0, n_s // 2, body, (), unroll=False)

    for b in range(2):
        for c in out_descs(n_s - 2 + b, b):
            c.wait()


def kernel(word_ids, tag_ids, word_table, tag_table):
    b, s = word_ids.shape
    comb = (word_ids.astype(jnp.int32)
            | (tag_ids.astype(jnp.int32) << TAG_SHIFT)).T
    tag_pad = jnp.pad(tag_table, ((0, 0), (0, TAG_PITCH - TAG_DIM))).reshape(-1)

    run = pl.kernel(
        _emb_kernel,
        out_type=jax.ShapeDtypeStruct((s, DBLK, NW, 8, CHUNK), jnp.float32),
        mesh=plsc.VectorSubcoreMesh(core_axis_name="c", subcore_axis_name="s"),
        compiler_params=pltpu.CompilerParams(needs_layout_passes=False),
        scratch_types=[
            pltpu.VMEM((s, CHUNK), jnp.int32),
            pltpu.VMEM((TAG_NUM * TAG_PITCH,), jnp.float32),
            [pltpu.VMEM((CHUNK, WORD_DIM), jnp.float32) for _ in range(2)],
            [pltpu.VMEM((OUT_DIM, CHUNK), jnp.float32) for _ in range(2)],
            [pltpu.VMEM((CHUNK,), jnp.int32) for _ in range(2)],
            [pltpu.SemaphoreType.DMA for _ in range(2)],
            [pltpu.SemaphoreType.DMA for _ in range(2)],
        ],
    )
    out = run(comb, word_table, tag_pad)
    out = out.transpose(2, 4, 0, 1, 3).reshape(b, s, OUT_DIM)
    return out
